# baseline (speedup 1.0000x reference)
import functools

import jax
import jax.numpy as jnp
from jax import lax
from jax.experimental import pallas as pl
from jax.experimental.pallas import tpu as pltpu
from jax.experimental.pallas import tpu_sc as plsc

_N = 10000
_E = 160000
_D = 256
_H = 512
_WC = 128
_NT = 16
_NC = 2
_B = 96
_EPT = 10080
_NB = _EPT // _B
_EPAD = _EPT * _NT
_NPAD = 10112
_RPT = _NPAD // _NT

_SC_MESH = dict(core_axis_name="c", subcore_axis_name="s",
                num_cores=_NC, num_subcores=_NT)


def _make_deg_kernel():
    @functools.partial(
        pl.kernel,
        out_type=jax.ShapeDtypeStruct((_NPAD, 16), jnp.float32),
        mesh=plsc.VectorSubcoreMesh(**_SC_MESH),
        scratch_types=[
            pltpu.VMEM((_NB, _B), jnp.int32),
            pltpu.VMEM((_B, 16), jnp.float32),
            pltpu.VMEM_SHARED((_NPAD, 16), jnp.float32),
            pltpu.SemaphoreType.DMA,
        ],
    )
    def deg_kernel(dstp_hbm, zeros_hbm, out_hbm, didx, ones, acc, sem):
        c = lax.axis_index("c")
        t = lax.axis_index("s")

        def fill(i, _):
            ones[i, :] = jnp.full((16,), 1.0, jnp.float32)
            return 0

        lax.fori_loop(0, _B, fill, 0)

        pltpu.sync_copy(zeros_hbm, acc.at[pl.ds(t * _RPT, _RPT)])
        pltpu.sync_copy(dstp_hbm.at[t], didx)
        plsc.subcore_barrier()

        def fire(i, _):
            pltpu.async_copy(ones, acc.at[didx.at[i]], sem, add=True)
            return 0

        lax.fori_loop(0, _NB, fire, 0)

        def drain(i, _):
            pltpu.make_async_copy(ones, acc.at[didx.at[0]], sem).wait()
            return 0

        lax.fori_loop(0, _NB, drain, 0)
        plsc.subcore_barrier()

        @pl.when(c == 0)
        def _():
            pltpu.sync_copy(acc.at[pl.ds(t * _RPT, _RPT)],
                            out_hbm.at[pl.ds(t * _RPT, _RPT)])

    return deg_kernel


def _make_agg_kernel(nch):
    npc = nch // _NC

    @functools.partial(
        pl.kernel,
        out_type=jax.ShapeDtypeStruct((nch * _NPAD, _WC), jnp.float32),
        mesh=plsc.VectorSubcoreMesh(**_SC_MESH),
        scratch_types=[
            pltpu.VMEM((_EPT,), jnp.int32),
            pltpu.VMEM((_NB, _B), jnp.int32),
            pltpu.VMEM((_B, _WC), jnp.float32),
            pltpu.VMEM((_B, _WC), jnp.float32),
            pltpu.VMEM_SHARED((_NPAD, _WC), jnp.float32),
            pltpu.SemaphoreType.DMA,
            pltpu.SemaphoreType.DMA,
            pltpu.SemaphoreType.DMA,
            pltpu.SemaphoreType.DMA,
        ],
    )
    def agg_kernel(hp_hbm, srcp_hbm, dstp_hbm, zeros_hbm, out_hbm,
                   sidx, didx, rows0, rows1, acc, gsem0, gsem1, ssem0, ssem1):
        c = lax.axis_index("c")
        t = lax.axis_index("s")
        pltpu.sync_copy(dstp_hbm.at[t], didx)

        def gat(i):
            return hp_hbm.at[sidx.at[pl.ds(i * _B, _B)]]

        for k in range(npc):
            cg = c * npc + k
            pltpu.sync_copy(zeros_hbm, acc.at[pl.ds(t * _RPT, _RPT)])
            pltpu.sync_copy(srcp_hbm.at[cg, t], sidx)
            plsc.subcore_barrier()

            def body(i, _):
                pltpu.async_copy(gat(i), rows0, gsem0).wait()
                pltpu.sync_copy(rows0, acc.at[didx.at[i]], add=True)
                return 0

            lax.fori_loop(0, _NB, body, 0)
            plsc.subcore_barrier()
            pltpu.sync_copy(
                acc.at[pl.ds(t * _RPT, _RPT)],
                out_hbm.at[pl.ds(cg * _NPAD + t * _RPT, _RPT)])

    return agg_kernel


_deg_call = _make_deg_kernel()
_agg_call_2 = _make_agg_kernel(2)
_agg_call_4 = _make_agg_kernel(4)


_R = 1000
_G = _N // _R

_SQRT_HALF = 0.7071067811865476


def _gelu(x):
    return 0.5 * x * (1.0 + lax.erf(x * _SQRT_HALF))


def _prep_body(degp_ref, x_ref, dinv_ref, cnt_ref, hp0_ref):
    d = jnp.sum(degp_ref[...], axis=-1, keepdims=True) * (1.0 / 16.0)
    dinv = lax.rsqrt(d + 1.0)
    dinv_ref[...] = dinv
    cnt_ref[...] = jnp.maximum(d, 1.0)
    xb = x_ref[...]
    for k in range(_D // _WC):
        hp0_ref[k] = xb[:, k * _WC:(k + 1) * _WC] * dinv


def _prep_call(degp, x):
    return pl.pallas_call(
        _prep_body,
        grid=(_G,),
        in_specs=[
            pl.BlockSpec((_R, 16), lambda i: (i, 0)),
            pl.BlockSpec((_R, _D), lambda i: (i, 0)),
        ],
        out_specs=[
            pl.BlockSpec((_R, 1), lambda i: (i, 0)),
            pl.BlockSpec((_R, 1), lambda i: (i, 0)),
            pl.BlockSpec((_D // _WC, _R, _WC), lambda i: (0, i, 0)),
        ],
        out_shape=[
            jax.ShapeDtypeStruct((_N, 1), jnp.float32),
            jax.ShapeDtypeStruct((_N, 1), jnp.float32),
            jax.ShapeDtypeStruct((_D // _WC, _N, _WC), jnp.float32),
        ],
    )(degp, x)


def _make_layer_call(nch_in, residual, scale_out):
    din = nch_in * _WC
    nch_out = _H // _WC

    def body(agg_ref, hin_ref, dinv_ref, w_ref, b_ref, g_ref, be_ref,
             hout_ref, hp_ref):
        aggv = agg_ref[...]
        zt = jnp.concatenate([aggv[k] for k in range(nch_in)], axis=-1)
        dv = dinv_ref[...]
        hin = hin_ref[...]
        z = dv * zt + (dv * dv) * hin
        y = lax.dot_general(z, w_ref[...], (((1,), (1,)), ((), ())),
                            preferred_element_type=jnp.float32) + b_ref[...]
        mu = jnp.mean(y, axis=-1, keepdims=True)
        yc = y - mu
        var = jnp.mean(yc * yc, axis=-1, keepdims=True)
        yn = yc * lax.rsqrt(var + 1e-5) * g_ref[...] + be_ref[...]
        a = _gelu(yn)
        hout = a + hin if residual else a
        hout_ref[...] = hout
        hp = hout * dv if scale_out else hout
        for k in range(nch_out):
            hp_ref[k] = hp[:, k * _WC:(k + 1) * _WC]

    def call(agg, hin, dinv, w, b, g, be):
        return pl.pallas_call(
            body,
            grid=(_G,),
            in_specs=[
                pl.BlockSpec((nch_in, _R, _WC), lambda i: (0, i, 0)),
                pl.BlockSpec((_R, din), lambda i: (i, 0)),
                pl.BlockSpec((_R, 1), lambda i: (i, 0)),
                pl.BlockSpec((_H, din), lambda i: (0, 0)),
                pl.BlockSpec((1, _H), lambda i: (0, 0)),
                pl.BlockSpec((1, _H), lambda i: (0, 0)),
                pl.BlockSpec((1, _H), lambda i: (0, 0)),
            ],
            out_specs=[
                pl.BlockSpec((_R, _H), lambda i: (i, 0)),
                pl.BlockSpec((nch_out, _R, _WC), lambda i: (0, i, 0)),
            ],
            out_shape=[
                jax.ShapeDtypeStruct((_N, _H), jnp.float32),
                jax.ShapeDtypeStruct((nch_out, _N, _WC), jnp.float32),
            ],
        )(agg, hin, dinv, w, b, g, be)

    return call


_layer0_call = _make_layer_call(_D // _WC, residual=False, scale_out=True)
_layer1_call = _make_layer_call(_H // _WC, residual=True, scale_out=True)
_layer2_call = _make_layer_call(_H // _WC, residual=True, scale_out=False)


def _colsum_body(agg_ref, h3_ref, cnt_ref, out_ref):
    @pl.when(pl.program_id(0) == 0)
    def _():
        out_ref[...] = jnp.zeros_like(out_ref)

    aggv = agg_ref[...]
    nch = aggv.shape[0]
    aggt = jnp.concatenate([aggv[k] for k in range(nch)], axis=-1)
    am = aggt / cnt_ref[...]
    s1 = jnp.sum(am, axis=0, keepdims=True)
    s2 = jnp.sum(h3_ref[...], axis=0, keepdims=True)
    out_ref[...] += jnp.concatenate([s1, s2], axis=0)


def _colsum_call(agg, h3, cnt):
    nch = _H // _WC
    return pl.pallas_call(
        _colsum_body,
        grid=(_G,),
        in_specs=[
            pl.BlockSpec((nch, _R, _WC), lambda i: (0, i, 0)),
            pl.BlockSpec((_R, _H), lambda i: (i, 0)),
            pl.BlockSpec((_R, 1), lambda i: (i, 0)),
        ],
        out_specs=pl.BlockSpec((2, _H), lambda i: (0, 0)),
        out_shape=jax.ShapeDtypeStruct((2, _H), jnp.float32),
    )(agg, h3, cnt)


def _final_body(u_ref, wl_ref, bl_ref, wr_ref, wc1_ref, bc1_ref, wc2_ref,
                bc2_ref, out_ref):
    u = u_ref[...] * (1.0 / _N)
    pooled = (
        lax.dot_general(u[0:1], wl_ref[...], (((1,), (1,)), ((), ())),
                        preferred_element_type=jnp.float32)
        + bl_ref[...]
        + lax.dot_general(u[1:2], wr_ref[...], (((1,), (1,)), ((), ())),
                          preferred_element_type=jnp.float32))
    tmp = lax.dot_general(pooled, wc1_ref[...], (((1,), (1,)), ((), ())),
                          preferred_element_type=jnp.float32) + bc1_ref[...]
    tmp = _gelu(tmp)
    out_ref[...] = lax.dot_general(tmp, wc2_ref[...], (((1,), (1,)), ((), ())),
                                   preferred_element_type=jnp.float32) + bc2_ref[...]


def _final_call(u, wl, bl, wr, wc1, bc1, wc2, bc2):
    return pl.pallas_call(
        _final_body,
        out_shape=jax.ShapeDtypeStruct((1, 2), jnp.float32),
    )(u, wl, bl, wr, wc1, bc1, wc2, bc2)


def kernel(x, edge_index, W0, b0, g0, be0, W1, b1, g1, be1, W2, b2, g2, be2,
           Wl, bl, Wr, Wc1, bc1, Wc2, bc2):
    npad_e = _EPAD - _E
    src = jnp.concatenate(
        [edge_index[0], jnp.zeros((npad_e,), jnp.int32)])
    dst = jnp.concatenate(
        [edge_index[1],
         _N + (jnp.arange(npad_e, dtype=jnp.int32) % (_NPAD - _N))])
    dstp = dst.reshape(_NT, _NB, _B)
    offs2 = (jnp.arange(2, dtype=jnp.int32) * _N)[:, None]
    offs4 = (jnp.arange(4, dtype=jnp.int32) * _N)[:, None]
    srcp2 = (src[None, :] + offs2).reshape(2, _NT, _EPT)
    srcp4 = (src[None, :] + offs4).reshape(4, _NT, _EPT)
    zeros16 = jnp.zeros((_RPT, 16), jnp.float32)
    zwc = jnp.zeros((_RPT, _WC), jnp.float32)

    degp = _deg_call(dstp, zeros16)
    dinv, cnt, hp0 = _prep_call(degp, x)

    agg0 = _agg_call_2(hp0.reshape(2 * _N, _WC), srcp2, dstp, zwc)
    h1, hp1 = _layer0_call(agg0.reshape(2, _NPAD, _WC), x, dinv, W0,
                           b0.reshape(1, _H), g0.reshape(1, _H),
                           be0.reshape(1, _H))

    agg1 = _agg_call_4(hp1.reshape(4 * _N, _WC), srcp4, dstp, zwc)
    h2, hp2 = _layer1_call(agg1.reshape(4, _NPAD, _WC), h1, dinv, W1,
                           b1.reshape(1, _H), g1.reshape(1, _H),
                           be1.reshape(1, _H))

    agg2 = _agg_call_4(hp2.reshape(4 * _N, _WC), srcp4, dstp, zwc)
    h3, hp3 = _layer2_call(agg2.reshape(4, _NPAD, _WC), h2, dinv, W2,
                           b2.reshape(1, _H), g2.reshape(1, _H),
                           be2.reshape(1, _H))

    aggs = _agg_call_4(hp3.reshape(4 * _N, _WC), srcp4, dstp, zwc)
    u = _colsum_call(aggs.reshape(4, _NPAD, _WC), h3, cnt)

    return _final_call(u, Wl, bl.reshape(1, _H), Wr, Wc1,
                       bc1.reshape(1, _H // 2), Wc2, bc2.reshape(1, 2))


# speedup vs baseline: 1.1637x; 1.1637x over previous
import functools

import jax
import jax.numpy as jnp
from jax import lax
from jax.experimental import pallas as pl
from jax.experimental.pallas import tpu as pltpu
from jax.experimental.pallas import tpu_sc as plsc

_N = 10000
_E = 160000
_D = 256
_H = 512
_WC = 128
_NT = 16
_NC = 2
_B = 80
_EPT = _E // _NT
_NB = _EPT // _B
_NPAD = 10240
_RPT = _NPAD // _NT

_SC_MESH = dict(core_axis_name="c", subcore_axis_name="s",
                num_cores=_NC, num_subcores=_NT)


def _make_deg_kernel():
    @functools.partial(
        pl.kernel,
        out_type=jax.ShapeDtypeStruct((_NPAD, 16), jnp.float32),
        mesh=plsc.VectorSubcoreMesh(**_SC_MESH),
        scratch_types=[
            pltpu.VMEM((_NB, _B), jnp.int32),
            pltpu.VMEM((_B, 16), jnp.float32),
            pltpu.VMEM_SHARED((_NPAD, 16), jnp.float32),
            pltpu.SemaphoreType.DMA,
        ],
    )
    def deg_kernel(dstp_hbm, zeros_hbm, out_hbm, didx, ones, acc, sem):
        c = lax.axis_index("c")
        t = lax.axis_index("s")

        def fill(i, _):
            ones[i, :] = jnp.full((16,), 1.0, jnp.float32)
            return 0

        lax.fori_loop(0, _B, fill, 0)

        pltpu.sync_copy(zeros_hbm, acc.at[pl.ds(t * _RPT, _RPT)])
        pltpu.sync_copy(dstp_hbm.at[t], didx)
        plsc.subcore_barrier()

        def fire(i, _):
            pltpu.async_copy(ones, acc.at[didx.at[i]], sem, add=True)
            return 0

        lax.fori_loop(0, _NB, fire, 0)

        def drain(i, _):
            pltpu.make_async_copy(ones, acc.at[didx.at[0]], sem).wait()
            return 0

        lax.fori_loop(0, _NB, drain, 0)
        plsc.subcore_barrier()

        @pl.when(c == 0)
        def _():
            pltpu.sync_copy(acc.at[pl.ds(t * _RPT, _RPT)],
                            out_hbm.at[pl.ds(t * _RPT, _RPT)])

    return deg_kernel


def _make_agg_kernel(nch):
    npc = nch // _NC

    @functools.partial(
        pl.kernel,
        out_type=jax.ShapeDtypeStruct((nch * _NPAD, _WC), jnp.float32),
        mesh=plsc.VectorSubcoreMesh(**_SC_MESH),
        scratch_types=[
            pltpu.VMEM((_EPT,), jnp.int32),
            pltpu.VMEM((_NB, _B), jnp.int32),
            pltpu.VMEM((_B, _WC), jnp.float32),
            pltpu.VMEM_SHARED((_NPAD, _WC), jnp.float32),
            pltpu.SemaphoreType.DMA,
        ],
    )
    def agg_kernel(hp_hbm, srcp_hbm, dstp_hbm, zeros_hbm, out_hbm,
                   sidx, didx, rows0, acc, gsem0):
        c = lax.axis_index("c")
        t = lax.axis_index("s")
        pltpu.sync_copy(dstp_hbm.at[t], didx)

        def gat(i):
            return hp_hbm.at[sidx.at[pl.ds(i * _B, _B)]]

        for k in range(npc):
            cg = c * npc + k
            pltpu.sync_copy(zeros_hbm, acc.at[pl.ds(t * _RPT, _RPT)])
            pltpu.sync_copy(srcp_hbm.at[cg, t], sidx)
            plsc.subcore_barrier()

            def body(i, _):
                pltpu.async_copy(gat(i), rows0, gsem0).wait()
                pltpu.sync_copy(rows0, acc.at[didx.at[i]], add=True)
                return 0

            lax.fori_loop(0, _NB, body, 0)
            plsc.subcore_barrier()
            pltpu.sync_copy(
                acc.at[pl.ds(t * _RPT, _RPT)],
                out_hbm.at[pl.ds(cg * _NPAD + t * _RPT, _RPT)])

    return agg_kernel


_deg_call = _make_deg_kernel()
_agg_call_2 = _make_agg_kernel(2)
_agg_call_4 = _make_agg_kernel(4)


_R = 1000
_G = _N // _R

_SQRT_HALF = 0.7071067811865476


def _gelu(x):
    return 0.5 * x * (1.0 + lax.erf(x * _SQRT_HALF))


def _prep_body(degp_ref, x_ref, dinv_ref, cnt_ref, hp0_ref):
    d = jnp.sum(degp_ref[...], axis=-1, keepdims=True) * (1.0 / 16.0)
    dinv = lax.rsqrt(d + 1.0)
    dinv_ref[...] = dinv
    cnt_ref[...] = jnp.maximum(d, 1.0)
    xb = x_ref[...]
    for k in range(_D // _WC):
        hp0_ref[k] = xb[:, k * _WC:(k + 1) * _WC] * dinv


def _prep_call(degp, x):
    return pl.pallas_call(
        _prep_body,
        grid=(_G,),
        in_specs=[
            pl.BlockSpec((_R, 16), lambda i: (i, 0)),
            pl.BlockSpec((_R, _D), lambda i: (i, 0)),
        ],
        out_specs=[
            pl.BlockSpec((_R, 1), lambda i: (i, 0)),
            pl.BlockSpec((_R, 1), lambda i: (i, 0)),
            pl.BlockSpec((_D // _WC, _R, _WC), lambda i: (0, i, 0)),
        ],
        out_shape=[
            jax.ShapeDtypeStruct((_N, 1), jnp.float32),
            jax.ShapeDtypeStruct((_N, 1), jnp.float32),
            jax.ShapeDtypeStruct((_D // _WC, _N, _WC), jnp.float32),
        ],
    )(degp, x)


def _make_layer_call(nch_in, residual, scale_out):
    din = nch_in * _WC
    nch_out = _H // _WC

    def body(agg_ref, hin_ref, dinv_ref, w_ref, b_ref, g_ref, be_ref,
             hout_ref, hp_ref):
        aggv = agg_ref[...]
        zt = jnp.concatenate([aggv[k] for k in range(nch_in)], axis=-1)
        dv = dinv_ref[...]
        hin = hin_ref[...]
        z = dv * zt + (dv * dv) * hin
        y = lax.dot_general(z, w_ref[...], (((1,), (1,)), ((), ())),
                            preferred_element_type=jnp.float32) + b_ref[...]
        mu = jnp.mean(y, axis=-1, keepdims=True)
        yc = y - mu
        var = jnp.mean(yc * yc, axis=-1, keepdims=True)
        yn = yc * lax.rsqrt(var + 1e-5) * g_ref[...] + be_ref[...]
        a = _gelu(yn)
        hout = a + hin if residual else a
        hout_ref[...] = hout
        hp = hout * dv if scale_out else hout
        for k in range(nch_out):
            hp_ref[k] = hp[:, k * _WC:(k + 1) * _WC]

    def call(agg, hin, dinv, w, b, g, be):
        return pl.pallas_call(
            body,
            grid=(_G,),
            in_specs=[
                pl.BlockSpec((nch_in, _R, _WC), lambda i: (0, i, 0)),
                pl.BlockSpec((_R, din), lambda i: (i, 0)),
                pl.BlockSpec((_R, 1), lambda i: (i, 0)),
                pl.BlockSpec((_H, din), lambda i: (0, 0)),
                pl.BlockSpec((1, _H), lambda i: (0, 0)),
                pl.BlockSpec((1, _H), lambda i: (0, 0)),
                pl.BlockSpec((1, _H), lambda i: (0, 0)),
            ],
            out_specs=[
                pl.BlockSpec((_R, _H), lambda i: (i, 0)),
                pl.BlockSpec((nch_out, _R, _WC), lambda i: (0, i, 0)),
            ],
            out_shape=[
                jax.ShapeDtypeStruct((_N, _H), jnp.float32),
                jax.ShapeDtypeStruct((nch_out, _N, _WC), jnp.float32),
            ],
        )(agg, hin, dinv, w, b, g, be)

    return call


_layer0_call = _make_layer_call(_D // _WC, residual=False, scale_out=True)
_layer1_call = _make_layer_call(_H // _WC, residual=True, scale_out=True)
_layer2_call = _make_layer_call(_H // _WC, residual=True, scale_out=False)


def _colsum_body(agg_ref, h3_ref, cnt_ref, out_ref):
    @pl.when(pl.program_id(0) == 0)
    def _():
        out_ref[...] = jnp.zeros_like(out_ref)

    aggv = agg_ref[...]
    nch = aggv.shape[0]
    aggt = jnp.concatenate([aggv[k] for k in range(nch)], axis=-1)
    am = aggt / cnt_ref[...]
    s1 = jnp.sum(am, axis=0, keepdims=True)
    s2 = jnp.sum(h3_ref[...], axis=0, keepdims=True)
    out_ref[...] += jnp.concatenate([s1, s2], axis=0)


def _colsum_call(agg, h3, cnt):
    nch = _H // _WC
    return pl.pallas_call(
        _colsum_body,
        grid=(_G,),
        in_specs=[
            pl.BlockSpec((nch, _R, _WC), lambda i: (0, i, 0)),
            pl.BlockSpec((_R, _H), lambda i: (i, 0)),
            pl.BlockSpec((_R, 1), lambda i: (i, 0)),
        ],
        out_specs=pl.BlockSpec((2, _H), lambda i: (0, 0)),
        out_shape=jax.ShapeDtypeStruct((2, _H), jnp.float32),
    )(agg, h3, cnt)


def _final_body(u_ref, wl_ref, bl_ref, wr_ref, wc1_ref, bc1_ref, wc2_ref,
                bc2_ref, out_ref):
    u = u_ref[...] * (1.0 / _N)
    pooled = (
        lax.dot_general(u[0:1], wl_ref[...], (((1,), (1,)), ((), ())),
                        preferred_element_type=jnp.float32)
        + bl_ref[...]
        + lax.dot_general(u[1:2], wr_ref[...], (((1,), (1,)), ((), ())),
                          preferred_element_type=jnp.float32))
    tmp = lax.dot_general(pooled, wc1_ref[...], (((1,), (1,)), ((), ())),
                          preferred_element_type=jnp.float32) + bc1_ref[...]
    tmp = _gelu(tmp)
    out_ref[...] = lax.dot_general(tmp, wc2_ref[...], (((1,), (1,)), ((), ())),
                                   preferred_element_type=jnp.float32) + bc2_ref[...]


def _final_call(u, wl, bl, wr, wc1, bc1, wc2, bc2):
    return pl.pallas_call(
        _final_body,
        out_shape=jax.ShapeDtypeStruct((1, 2), jnp.float32),
    )(u, wl, bl, wr, wc1, bc1, wc2, bc2)


def kernel(x, edge_index, W0, b0, g0, be0, W1, b1, g1, be1, W2, b2, g2, be2,
           Wl, bl, Wr, Wc1, bc1, Wc2, bc2):
    src = edge_index[0]
    dst = edge_index[1]
    dstp = dst.reshape(_NT, _NB, _B)
    offs2 = (jnp.arange(2, dtype=jnp.int32) * _N)[:, None]
    offs4 = (jnp.arange(4, dtype=jnp.int32) * _N)[:, None]
    srcp2 = (src[None, :] + offs2).reshape(2, _NT, _EPT)
    srcp4 = (src[None, :] + offs4).reshape(4, _NT, _EPT)
    zeros16 = jnp.zeros((_RPT, 16), jnp.float32)
    zwc = jnp.zeros((_RPT, _WC), jnp.float32)

    degp = _deg_call(dstp, zeros16)
    dinv, cnt, hp0 = _prep_call(degp, x)

    agg0 = _agg_call_2(hp0.reshape(2 * _N, _WC), srcp2, dstp, zwc)
    h1, hp1 = _layer0_call(agg0.reshape(2, _NPAD, _WC), x, dinv, W0,
                           b0.reshape(1, _H), g0.reshape(1, _H),
                           be0.reshape(1, _H))

    agg1 = _agg_call_4(hp1.reshape(4 * _N, _WC), srcp4, dstp, zwc)
    h2, hp2 = _layer1_call(agg1.reshape(4, _NPAD, _WC), h1, dinv, W1,
                           b1.reshape(1, _H), g1.reshape(1, _H),
                           be1.reshape(1, _H))

    agg2 = _agg_call_4(hp2.reshape(4 * _N, _WC), srcp4, dstp, zwc)
    h3, hp3 = _layer2_call(agg2.reshape(4, _NPAD, _WC), h2, dinv, W2,
                           b2.reshape(1, _H), g2.reshape(1, _H),
                           be2.reshape(1, _H))

    aggs = _agg_call_4(hp3.reshape(4 * _N, _WC), srcp4, dstp, zwc)
    u = _colsum_call(aggs.reshape(4, _NPAD, _WC), h3, cnt)

    return _final_call(u, Wl, bl.reshape(1, _H), Wr, Wc1,
                       bc1.reshape(1, _H // 2), Wc2, bc2.reshape(1, 2))
